# baseline (speedup 1.0000x reference)
import functools

import jax
import jax.numpy as jnp
from jax import lax
from jax.experimental import pallas as pl
from jax.experimental.pallas import tpu as pltpu, tpu_sc as plsc

_N_SLOTS = 64
_EMBED_DIM = 256
_N_CODES = 8192
_BETA = 0.25
_BATCH = 8
_N_WORKERS = 8
_ROWS_PER_WORKER = _N_SLOTS // _N_WORKERS


def _distance_argmin_kernel(slots_ref, W_enc_ref, b_enc_ref, W_prev_ref,
                            b_prev_ref, cb_ref, s_ref, idx_ref):
    f32 = jnp.float32
    h = jnp.maximum(
        jnp.dot(slots_ref[...], W_enc_ref[...], preferred_element_type=f32)
        + b_enc_ref[...], 0.0)
    s = (jnp.dot(h, W_prev_ref[...], preferred_element_type=f32)
         + b_prev_ref[...])
    cb = cb_ref[...]
    a = jnp.sum(s * s, axis=1, keepdims=True)
    m = jax.lax.dot_general(s, cb, (((1,), (1,)), ((), ())),
                            preferred_element_type=f32)
    cn = jnp.sum(cb * cb, axis=1)
    d = (a - 2.0 * m) + cn[None, :]
    dmin = jnp.min(d, axis=1, keepdims=True)
    col = jax.lax.broadcasted_iota(jnp.int32, d.shape, 1)
    big = jnp.int32(jnp.iinfo(jnp.int32).max)
    idx = jnp.min(jnp.where(d == dmin, col, big), axis=1)
    s_ref[...] = s
    idx_ref[...] = jnp.reshape(idx, (1, _N_SLOTS))


def _decode_kernel(s_ref, zq_ref, W_post_ref, b_post_ref, W_dec_ref,
                   b_dec_ref, rec_ref, loss_ref):
    f32 = jnp.float32
    s = s_ref[...]
    zq = zq_ref[...]
    diff = zq - s
    loss = (1.0 + _BETA) * jnp.sum(diff * diff) / (_N_SLOTS * _EMBED_DIM)
    loss_ref[...] = jnp.reshape(loss, (1, 1))
    dec_in = (jnp.dot(zq, W_post_ref[...], preferred_element_type=f32)
              + b_post_ref[...])
    rec = (jnp.dot(dec_in, W_dec_ref[...], preferred_element_type=f32)
           + b_dec_ref[...])
    rec = jnp.clip(rec, -1.0, 1.0)
    for b in range(_BATCH):
        rec_ref[b] = rec


def _sc_gather(idx_hbm, table_hbm, out_hbm, idx_v, rows_v, sem):
    wid = lax.axis_index("s") * 2 + lax.axis_index("c")

    @pl.when(wid < _N_WORKERS)
    def _():
        base = wid * _ROWS_PER_WORKER
        pltpu.sync_copy(idx_hbm.at[0, pl.ds(base, _ROWS_PER_WORKER)], idx_v)
        pltpu.async_copy(table_hbm.at[idx_v], rows_v, sem).wait()
        pltpu.sync_copy(rows_v, out_hbm.at[pl.ds(base, _ROWS_PER_WORKER), :])


_sc_gather_call = functools.partial(
    pl.kernel,
    out_type=jax.ShapeDtypeStruct((_N_SLOTS, _EMBED_DIM), jnp.float32),
    mesh=plsc.VectorSubcoreMesh(core_axis_name="c", subcore_axis_name="s"),
    scratch_types=[
        pltpu.VMEM((_ROWS_PER_WORKER,), jnp.int32),
        pltpu.VMEM((_ROWS_PER_WORKER, _EMBED_DIM), jnp.float32),
        pltpu.SemaphoreType.DMA,
    ],
)(_sc_gather)


def kernel(img, targets, slots, W_enc, b_enc, W_prev, b_prev, codebook,
           W_post, b_post, W_dec, b_dec):
    bs = img.shape[0]
    enc_dim = W_dec.shape[1]
    f32 = jnp.float32
    s, idx = pl.pallas_call(
        _distance_argmin_kernel,
        out_shape=[
            jax.ShapeDtypeStruct((_N_SLOTS, _EMBED_DIM), f32),
            jax.ShapeDtypeStruct((1, _N_SLOTS), jnp.int32),
        ],
    )(slots, W_enc, b_enc.reshape(1, -1), W_prev, b_prev.reshape(1, -1),
      codebook)
    zq = _sc_gather_call(idx, codebook)
    rec, loss = pl.pallas_call(
        _decode_kernel,
        out_shape=[
            jax.ShapeDtypeStruct((bs, _N_SLOTS, enc_dim), f32),
            jax.ShapeDtypeStruct((1, 1), f32),
        ],
    )(s, zq, W_post, b_post.reshape(1, -1), W_dec, b_dec.reshape(1, -1))
    q_indices = jnp.broadcast_to(idx, (bs, _N_SLOTS))
    return rec, jnp.reshape(loss, ()), q_indices


# speedup vs baseline: 2.7916x; 2.7916x over previous
import jax
import jax.numpy as jnp
from jax.experimental import pallas as pl

_N_SLOTS = 64
_EMBED_DIM = 256
_N_CODES = 8192
_BETA = 0.25
_BATCH = 8


def _fused_vq_kernel(slots_ref, W_enc_ref, b_enc_ref, W_prev_ref, b_prev_ref,
                     cb1_ref, cb2_ref, W_post_ref, b_post_ref, W_dec_ref,
                     b_dec_ref, rec_ref, loss_ref, idx_ref):
    f32 = jnp.float32
    h = jnp.maximum(
        jnp.dot(slots_ref[...], W_enc_ref[...], preferred_element_type=f32)
        + b_enc_ref[...], 0.0)
    s = (jnp.dot(h, W_prev_ref[...], preferred_element_type=f32)
         + b_prev_ref[...])
    cb = jnp.concatenate([cb1_ref[...], cb2_ref[...]], axis=0)
    a = jnp.sum(s * s, axis=1, keepdims=True)
    m = jax.lax.dot_general(s, cb, (((1,), (1,)), ((), ())),
                            preferred_element_type=f32)
    cn = jnp.sum(cb * cb, axis=1)
    d = (a - 2.0 * m) + cn[None, :]
    dmin = jnp.min(d, axis=1, keepdims=True)
    col = jax.lax.broadcasted_iota(jnp.int32, d.shape, 1)
    big = jnp.int32(jnp.iinfo(jnp.int32).max)
    idx = jnp.min(jnp.where(d == dmin, col, big), axis=1)
    onehot = (col == idx[:, None]).astype(f32)
    zq = jnp.dot(onehot, cb, preferred_element_type=f32)
    diff = zq - s
    loss = (1.0 + _BETA) * jnp.sum(diff * diff) / (_N_SLOTS * _EMBED_DIM)
    loss_ref[...] = jnp.reshape(loss, (1, 1))
    dec_in = (jnp.dot(zq, W_post_ref[...], preferred_element_type=f32)
              + b_post_ref[...])
    rec = (jnp.dot(dec_in, W_dec_ref[...], preferred_element_type=f32)
           + b_dec_ref[...])
    rec = jnp.clip(rec, -1.0, 1.0)
    for b in range(_BATCH):
        rec_ref[b] = rec
    idx_ref[...] = jnp.broadcast_to(idx[None, :], (_BATCH, _N_SLOTS))


def kernel(img, targets, slots, W_enc, b_enc, W_prev, b_prev, codebook,
           W_post, b_post, W_dec, b_dec):
    bs = img.shape[0]
    enc_dim = W_dec.shape[1]
    half = _N_CODES // 2
    whole = lambda arr: pl.BlockSpec(arr.shape, lambda i: (0,) * arr.ndim)
    rec, loss, idx = pl.pallas_call(
        _fused_vq_kernel,
        grid=(1,),
        in_specs=[
            whole(slots), whole(W_enc), pl.BlockSpec((1, 512), lambda i: (0, 0)),
            whole(W_prev), pl.BlockSpec((1, 256), lambda i: (0, 0)),
            pl.BlockSpec((half, _EMBED_DIM), lambda i: (0, 0)),
            pl.BlockSpec((half, _EMBED_DIM), lambda i: (1, 0)),
            whole(W_post), pl.BlockSpec((1, 512), lambda i: (0, 0)),
            whole(W_dec), pl.BlockSpec((1, 512), lambda i: (0, 0)),
        ],
        out_specs=[
            pl.BlockSpec((bs, _N_SLOTS, enc_dim), lambda i: (0, 0, 0)),
            pl.BlockSpec((1, 1), lambda i: (0, 0)),
            pl.BlockSpec((bs, _N_SLOTS), lambda i: (0, 0)),
        ],
        out_shape=[
            jax.ShapeDtypeStruct((bs, _N_SLOTS, enc_dim), jnp.float32),
            jax.ShapeDtypeStruct((1, 1), jnp.float32),
            jax.ShapeDtypeStruct((bs, _N_SLOTS), jnp.int32),
        ],
    )(slots, W_enc, b_enc.reshape(1, -1), W_prev, b_prev.reshape(1, -1),
      codebook, codebook, W_post, b_post.reshape(1, -1), W_dec,
      b_dec.reshape(1, -1))
    return rec, jnp.reshape(loss, ()), idx


# speedup vs baseline: 2.8149x; 1.0083x over previous
import jax
import jax.numpy as jnp
from jax.experimental import pallas as pl

_N_SLOTS = 64
_EMBED_DIM = 256
_N_CODES = 8192
_BETA = 0.25
_BATCH = 8


def _fused_vq_kernel(slots_ref, W_enc_ref, b_enc_ref, W_prev_ref, b_prev_ref,
                     cb_ref, W_post_ref, b_post_ref, W_dec_ref, b_dec_ref,
                     rec_ref, loss_ref, idx_ref):
    f32 = jnp.float32
    h = jnp.maximum(
        jnp.dot(slots_ref[...], W_enc_ref[...], preferred_element_type=f32)
        + b_enc_ref[...], 0.0)
    s = (jnp.dot(h, W_prev_ref[...], preferred_element_type=f32)
         + b_prev_ref[...])
    cb = cb_ref[...]
    a = jnp.sum(s * s, axis=1, keepdims=True)
    m = jax.lax.dot_general(s, cb, (((1,), (1,)), ((), ())),
                            preferred_element_type=f32)
    cn = jnp.sum(cb * cb, axis=1)
    d = (a - 2.0 * m) + cn[None, :]
    dmin = jnp.min(d, axis=1, keepdims=True)
    col = jax.lax.broadcasted_iota(jnp.int32, d.shape, 1)
    big = jnp.int32(jnp.iinfo(jnp.int32).max)
    idx = jnp.min(jnp.where(d == dmin, col, big), axis=1)
    onehot = (col == idx[:, None]).astype(f32)
    zq = jnp.dot(onehot, cb, preferred_element_type=f32)
    diff = zq - s
    loss = (1.0 + _BETA) * jnp.sum(diff * diff) / (_N_SLOTS * _EMBED_DIM)
    loss_ref[...] = jnp.reshape(loss, (1, 1))
    dec_in = (jnp.dot(zq, W_post_ref[...], preferred_element_type=f32)
              + b_post_ref[...])
    rec = (jnp.dot(dec_in, W_dec_ref[...], preferred_element_type=f32)
           + b_dec_ref[...])
    rec = jnp.clip(rec, -1.0, 1.0)
    for b in range(_BATCH):
        rec_ref[b] = rec
    idx_ref[...] = jnp.broadcast_to(idx[None, :], (_BATCH, _N_SLOTS))


def kernel(img, targets, slots, W_enc, b_enc, W_prev, b_prev, codebook,
           W_post, b_post, W_dec, b_dec):
    bs = img.shape[0]
    enc_dim = W_dec.shape[1]
    rec, loss, idx = pl.pallas_call(
        _fused_vq_kernel,
        out_shape=[
            jax.ShapeDtypeStruct((bs, _N_SLOTS, enc_dim), jnp.float32),
            jax.ShapeDtypeStruct((1, 1), jnp.float32),
            jax.ShapeDtypeStruct((bs, _N_SLOTS), jnp.int32),
        ],
    )(slots, W_enc, b_enc.reshape(1, -1), W_prev, b_prev.reshape(1, -1),
      codebook, W_post, b_post.reshape(1, -1), W_dec, b_dec.reshape(1, -1))
    return rec, jnp.reshape(loss, ()), idx
